# baseline (speedup 1.0000x reference)
import jax
import jax.numpy as jnp
from jax.experimental import pallas as pl


def _mlp2(x, p):
    h = jax.nn.relu(x @ p['W1'] + p['b1'])
    return h @ p['W2'] + p['b2']


def _gate(x, p):
    a = _mlp2(x, p)
    return jax.nn.softmax(a, axis=1) * x * x.shape[1]


def _enconv(x, src, dst, ea, p):
    msg = _mlp2(jnp.concatenate([x[src], ea], axis=1), p)
    agg = jax.ops.segment_sum(msg, dst, num_segments=x.shape[0])
    return jax.nn.relu(x @ p['Wr'] + p['br'] + agg)


def _edge_update(x, src, dst, ea, p):
    z = jnp.concatenate([x[src], x[dst], ea], axis=1)
    return jax.nn.relu(_mlp2(z, p))


def _seg_softmax(e, seg, nseg):
    m = jax.ops.segment_max(e, seg, num_segments=nseg)
    m = jnp.where(jnp.isfinite(m), m, 0.0)
    w = jnp.exp(e - m[seg])
    s = jax.ops.segment_sum(w, seg, num_segments=nseg)
    return w / (s[seg] + 1e-16)


def _set2set(x, seg, p, nseg, steps=3):
    d = x.shape[1]
    h = jnp.zeros((nseg, d), x.dtype)
    c = jnp.zeros((nseg, d), x.dtype)
    q_star = jnp.zeros((nseg, 2 * d), x.dtype)
    for _ in range(steps):
        g = q_star @ p['Wih'] + p['bih'] + h @ p['Whh'] + p['bhh']
        i, f, gg, o = jnp.split(g, 4, axis=1)
        c = jax.nn.sigmoid(f) * c + jax.nn.sigmoid(i) * jnp.tanh(gg)
        h = jax.nn.sigmoid(o) * jnp.tanh(c)
        e = jnp.sum(x * h[seg], axis=1)
        a = _seg_softmax(e, seg, nseg)
        r = jax.ops.segment_sum(a[:, None] * x, seg, num_segments=nseg)
        q_star = jnp.concatenate([h, r], axis=1)
    return q_star


def _gmp(x, seg, nseg):
    m = jax.ops.segment_max(x, seg, num_segments=nseg)
    return jnp.where(jnp.isfinite(m), m, 0.0)


def _gap(x, seg, nseg):
    s = jax.ops.segment_sum(x, seg, num_segments=nseg)
    cnt = jax.ops.segment_sum(jnp.ones((x.shape[0],), x.dtype), seg, num_segments=nseg)
    return s / jnp.clip(cnt, 1.0, None)[:, None]


def _copy_kernel(a_ref, o_ref):
    o_ref[...] = a_ref[...]


def kernel(x, edge_attr, params, edge_index, x_batch, edge_attr_batch):
    B = 128
    src, dst = edge_index[0], edge_index[1]
    x = _enconv(x, src, dst, edge_attr, params['layer1'])
    ea = _edge_update(x, src, dst, edge_attr, params['eu1'])
    x = _gate(x, params['att_node1'])
    ea = _gate(ea, params['att_edge1'])
    x = _enconv(x, src, dst, ea, params['layer2'])
    ea = _edge_update(x, src, dst, ea, params['eu2'])
    x = _gate(x, params['att_node2'])
    ea2 = _gate(ea, params['att_edge2'])
    x = _enconv(x, src, dst, ea2, params['layer3'])
    ea = _edge_update(x, src, dst, ea2, params['eu3'])
    x = _gate(x, params['att_node3'])
    ea = _gate(ea, params['att_edge3'])
    x = _enconv(x, src, dst, ea, params['layer4'])
    ea = _edge_update(x, src, dst, ea, params['eu4'])
    x_out = _set2set(x, x_batch, params['set2set'], B)
    ea_out = _set2set(ea, edge_attr_batch, params['set2set'], B)
    pool_node = jnp.concatenate([_gmp(x, x_batch, B), _gap(x, x_batch, B)], axis=1)
    pool_edge = jnp.concatenate([_gmp(ea, edge_attr_batch, B), _gap(ea, edge_attr_batch, B)], axis=1)
    out = jnp.concatenate([x_out, ea_out, pool_node, pool_edge], axis=1)
    out = jax.nn.softmax(_mlp2(out, params['att']), axis=1) * out * out.shape[1]
    out = jax.nn.relu(out @ params['lin1']['W'] + params['lin1']['b'])
    out = out @ params['lin2']['W'] + params['lin2']['b']
    out = pl.pallas_call(
        _copy_kernel,
        out_shape=jax.ShapeDtypeStruct(out.shape, out.dtype),
    )(out)
    return out


# speedup vs baseline: 1.1061x; 1.1061x over previous
import functools

import jax
import jax.numpy as jnp
from jax import lax
from jax.experimental import pallas as pl
from jax.experimental.pallas import tpu as pltpu
from jax.experimental.pallas import tpu_sc as plsc

_INTERPRET = False

N, E, B = 10000, 160000, 128
NP = 10240
EP = 163840
RN = 2048
RE = 2048
NEG = -jnp.inf
_HI = jax.lax.Precision.HIGHEST


def _sum64(v):
    acc = v[:, 0:8]
    for r in range(1, 8):
        acc = acc + v[:, 8 * r:8 * r + 8]
    acc = acc[:, 0:4] + acc[:, 4:8]
    acc = acc[:, 0:2] + acc[:, 2:4]
    return acc[:, 0:1] + acc[:, 1:2]


def _softmax64(a):
    e = jnp.exp(a - jnp.max(a, axis=1, keepdims=True))
    return e / _sum64(e)


def _pcall(body, grid, in_specs, out_specs, out_shape):
    return pl.pallas_call(
        body, grid=grid, in_specs=in_specs, out_specs=out_specs,
        out_shape=out_shape, interpret=_INTERPRET)


def _full(shape):
    return pl.BlockSpec(shape, lambda i: (0,) * len(shape))


def _rows(bs, w):
    return pl.BlockSpec((bs, w), lambda i: (i, 0))


def _msg1_body(xs_ref, ea_ref, w1, b1, w2, b2, o_ref):
    z = jnp.concatenate([xs_ref[:, 0:16], ea_ref[...]], axis=1)
    h = z @ w1[...] + b1[...]
    m = jax.nn.relu(h) @ w2[...] + b2[...]
    row = pl.program_id(0) * RE + jax.lax.broadcasted_iota(jnp.int32, (RE, 1), 0)
    o_ref[...] = jnp.where(row < E, m, 0.0)


def _msg1(xs, ea, w1, b1, w2, b2):
    grid = (EP // RE,)
    return _pcall(
        _msg1_body, grid,
        [_rows(RE, 128), _rows(RE, 8), _full((24, 128)),
         _full((1, 128)), _full((128, 64)), _full((1, 64))],
        _rows(RE, 64),
        jax.ShapeDtypeStruct((EP, 64), jnp.float32),
    )(xs, ea, w1, b1, w2, b2)


def _nodeup_body(gate, xin_ref, agg_ref, wr, br, g1, gb1, g2, gb2,
                 x_ref, xg_ref=None):
    agg = jnp.sum(agg_ref[...], axis=0)
    xn = jax.nn.relu(xin_ref[...] @ wr[...] + br[...] + agg)
    x_ref[...] = xn
    if gate:
        a = jax.nn.relu(xn @ g1[...] + gb1[...]) @ g2[...] + gb2[...]
        xg_ref[...] = _softmax64(a) * xn * 64.0


def _node_update(xin, aggs, wr, br, gp=None):
    din = xin.shape[1]
    P = aggs.shape[0]
    gate = gp is not None
    grid = (NP // RN,)
    in_specs = [_rows(RN, din),
                pl.BlockSpec((P, RN, 64), lambda i: (0, i, 0)),
                _full(wr.shape), _full((1, 64))]
    args = [xin, aggs, wr, br]
    if gate:
        in_specs += [_full((64, 256)), _full((1, 256)),
                     _full((256, 64)), _full((1, 64))]
        args += [gp['W1'], gp['b1'].reshape(1, -1), gp['W2'], gp['b2'].reshape(1, -1)]
        out_specs = (_rows(RN, 64), _rows(RN, 64))
        out_shape = (jax.ShapeDtypeStruct((NP, 64), jnp.float32),
                     jax.ShapeDtypeStruct((NP, 64), jnp.float32))
        body = functools.partial(_nodeup_body, True)
    else:
        out_specs = _rows(RN, 64)
        out_shape = jax.ShapeDtypeStruct((NP, 64), jnp.float32)
        def body(xin_ref, agg_ref, wr_, br_, x_ref):
            _nodeup_body(False, xin_ref, agg_ref, wr_, br_, None, None, None,
                         None, x_ref)
    return _pcall(body, grid, in_specs, out_specs, out_shape)(*args)


def _edge_round_body(dea, gs_ref, gd_ref, ea_ref,
                     w1, b1, w2, b2,
                     g1, gb1, g2, gb2,
                     m1, mb1, m2, mb2,
                     eag_ref, msg_ref):
    z = jnp.concatenate([gs_ref[:, 0:64], gd_ref[:, 0:64], ea_ref[...]], axis=1)
    z = z @ w1[...] + b1[...]
    eak = jax.nn.relu(jax.nn.relu(z) @ w2[...] + b2[...])
    t = jax.nn.relu(eak @ g1[...] + gb1[...]) @ g2[...] + gb2[...]
    eag = _softmax64(t) * eak * 64.0
    eag_ref[...] = eag
    zm = jnp.concatenate([gs_ref[:, 64:128], eag], axis=1)
    h = jax.nn.relu(zm @ m1[...] + mb1[...])
    m = h @ m2[...] + mb2[...]
    row = pl.program_id(0) * RE + jax.lax.broadcasted_iota(jnp.int32, (RE, 1), 0)
    msg_ref[...] = jnp.where(row < E, m, 0.0)


def _edge_round(gs, gd, ea, eup, gp, mp_w1, mp_b1, mp_w2, mp_b2):
    dea = ea.shape[1]
    kdim = 128 + dea
    w1 = jnp.zeros((kdim, 128), jnp.float32).at[:eup['W1'].shape[0]].set(eup['W1'])
    grid = (EP // RE,)
    in_specs = [_rows(RE, 128), _rows(RE, 128), _rows(RE, dea),
                _full((kdim, 128)),
                _full((1, 128)), _full((128, 64)), _full((1, 64)),
                _full((64, 256)), _full((1, 256)), _full((256, 64)), _full((1, 64)),
                _full((128, 512)), _full((1, 512)),
                _full((512, 64)), _full((1, 64))]
    out_specs = (_rows(RE, 64), _rows(RE, 64))
    out_shape = (jax.ShapeDtypeStruct((EP, 64), jnp.float32),
                 jax.ShapeDtypeStruct((EP, 64), jnp.float32))
    return _pcall(functools.partial(_edge_round_body, dea), grid,
                  in_specs, out_specs, out_shape)(
        gs, gd, ea, w1, eup['b1'].reshape(1, -1),
        eup['W2'], eup['b2'].reshape(1, -1),
        gp['W1'], gp['b1'].reshape(1, -1), gp['W2'], gp['b2'].reshape(1, -1),
        mp_w1, mp_b1, mp_w2, mp_b2)


def _edge_final_body(dea, gs_ref, gd_ref, ea_ref, w1, b1, w2, b2, o_ref):
    z = jnp.concatenate([gs_ref[:, 0:64], gd_ref[:, 0:64], ea_ref[...]], axis=1)
    z = z @ w1[...] + b1[...]
    o_ref[...] = jax.nn.relu(jax.nn.relu(z) @ w2[...] + b2[...])


def _edge_final(gs, gd, ea, eup):
    dea = ea.shape[1]
    kdim = 128 + dea
    grid = (EP // RE,)
    in_specs = [_rows(RE, 128), _rows(RE, 128), _rows(RE, dea),
                _full((kdim, 128)),
                _full((1, 128)), _full((128, 64)), _full((1, 64))]
    return _pcall(functools.partial(_edge_final_body, dea), grid, in_specs,
                  _rows(RE, 64), jax.ShapeDtypeStruct((EP, 64), jnp.float32))(
        gs, gd, ea, eup['W1'], eup['b1'].reshape(1, -1), eup['W2'],
        eup['b2'].reshape(1, -1))


def _pool_body(bs, x_ref, seg_ref, gmp_ref):
    i = pl.program_id(0)
    x = x_ref[...]
    seg = seg_ref[...]

    @pl.when(i == 0)
    def _():
        gmp_ref[...] = jnp.full((B, 64), NEG, jnp.float32)

    def body(b, acc):
        mb = (seg[:, 0] == b)
        pm = jnp.max(jnp.where(mb[:, None], x, NEG), axis=0)
        oh = (jax.lax.broadcasted_iota(jnp.int32, (B, 1), 0) == b)
        return jnp.maximum(acc, jnp.where(oh, pm[None, :], NEG))
    acc = jax.lax.fori_loop(0, B, body, jnp.full((B, 64), NEG, jnp.float32))
    gmp_ref[...] = jnp.maximum(gmp_ref[...], acc)

    @pl.when(i == pl.num_programs(0) - 1)
    def _():
        g = gmp_ref[...]
        gmp_ref[...] = jnp.where(jnp.isfinite(g), g, 0.0)


def _pool_stats(x, seg, bs):
    rows = x.shape[0]
    grid = (rows // bs,)
    return _pcall(functools.partial(_pool_body, bs), grid,
                  [_rows(bs, 64), _rows(bs, 1)],
                  pl.BlockSpec((B, 64), lambda i: (0, 0)),
                  jax.ShapeDtypeStruct((B, 64), jnp.float32))(x, seg)


def _gap(x, seg1, nrows):
    s = jax.ops.segment_sum(x[:nrows], seg1[:nrows], num_segments=B)
    cnt = jax.ops.segment_sum(jnp.ones((nrows,), jnp.float32), seg1[:nrows],
                              num_segments=B)
    return s / jnp.clip(cnt, 1.0, None)[:, None]


def _s2s1_body(bs, x_ref, seg_ref, h_ref, e_ref, m_ref):
    i = pl.program_id(0)
    x = x_ref[...]
    seg = seg_ref[...]
    maskf = (seg == jax.lax.broadcasted_iota(jnp.int32, (bs, B), 1)).astype(jnp.float32)
    hsel = jax.lax.dot_general(maskf, h_ref[...], (((1,), (0,)), ((), ())),
                               precision=_HI,
                               preferred_element_type=jnp.float32)
    e = _sum64(x * hsel)
    e_ref[...] = e
    pm = jnp.max(jnp.where(maskf > 0, e, NEG), axis=0)

    @pl.when(i == 0)
    def _():
        m_ref[...] = jnp.full((1, B), NEG, jnp.float32)
    m_ref[...] = jnp.maximum(m_ref[...], pm[None, :])


def _s2s_pass1(x, seg, h, bs):
    rows = x.shape[0]
    grid = (rows // bs,)
    return _pcall(functools.partial(_s2s1_body, bs), grid,
                  [_rows(bs, 64), _rows(bs, 1), _full((B, 64))],
                  (_rows(bs, 1), pl.BlockSpec((1, B), lambda i: (0, 0))),
                  (jax.ShapeDtypeStruct((rows, 1), jnp.float32),
                   jax.ShapeDtypeStruct((1, B), jnp.float32)))(x, seg, h)


def _s2sw_body(bs, seg_ref, e_ref, m_ref, w_ref):
    seg = seg_ref[...]
    maskf = (seg == jax.lax.broadcasted_iota(jnp.int32, (bs, B), 1)).astype(jnp.float32)
    m = m_ref[...]
    m = jnp.where(jnp.isfinite(m), m, 0.0)
    msel = jnp.sum(maskf * m, axis=1, keepdims=True)
    w_ref[...] = jnp.exp(e_ref[...] - msel)


def _s2s_w(seg, e, m, bs):
    rows = seg.shape[0]
    grid = (rows // bs,)
    return _pcall(functools.partial(_s2sw_body, bs), grid,
                  [_rows(bs, 1), _rows(bs, 1),
                   pl.BlockSpec((1, B), lambda i: (0, 0))],
                  _rows(bs, 1),
                  jax.ShapeDtypeStruct((rows, 1), jnp.float32))(seg, e, m)


def _s2sax_body(bs, x_ref, seg_ref, w_ref, s_ref, ax_ref):
    seg = seg_ref[...]
    maskf = (seg == jax.lax.broadcasted_iota(jnp.int32, (bs, B), 1)).astype(jnp.float32)
    ssel = jnp.sum(maskf * s_ref[...], axis=1, keepdims=True)
    a = w_ref[...] / (ssel + 1e-16)
    ax_ref[...] = a * x_ref[...]


def _s2s_ax(x, seg, w, s, bs):
    rows = x.shape[0]
    grid = (rows // bs,)
    return _pcall(functools.partial(_s2sax_body, bs), grid,
                  [_rows(bs, 64), _rows(bs, 1), _rows(bs, 1),
                   pl.BlockSpec((1, B), lambda i: (0, 0))],
                  _rows(bs, 64),
                  jax.ShapeDtypeStruct((rows, 64), jnp.float32))(x, seg, w, s)


def _lstm_body(first, r_ref, h_ref, c_ref, wih, bih, whh, bhh,
               ho_ref, co_ref, qo_ref):
    h = h_ref[...]
    c = c_ref[...]
    if first:
        q = jnp.zeros((B, 128), jnp.float32)
    else:
        q = jnp.concatenate([h, r_ref[...]], axis=1)
    qo_ref[...] = q
    g = q @ wih[...] + bih[...] + h @ whh[...] + bhh[...]
    i = g[:, 0:64]
    f = g[:, 64:128]
    gg = g[:, 128:192]
    o = g[:, 192:256]
    cn = jax.nn.sigmoid(f) * c + jax.nn.sigmoid(i) * jnp.tanh(gg)
    hn = jax.nn.sigmoid(o) * jnp.tanh(cn)
    ho_ref[...] = hn
    co_ref[...] = cn


def _s2s_lstm(r, h, c, p, first):
    in_specs = [_full((B, 64)), _full((B, 64)), _full((B, 64)),
                _full((128, 256)), _full((1, 256)), _full((64, 256)),
                _full((1, 256))]
    out_specs = (_full((B, 64)), _full((B, 64)), _full((B, 128)))
    out_shape = (jax.ShapeDtypeStruct((B, 64), jnp.float32),
                 jax.ShapeDtypeStruct((B, 64), jnp.float32),
                 jax.ShapeDtypeStruct((B, 128), jnp.float32))
    return _pcall(functools.partial(_lstm_body, first), (1,), in_specs,
                  out_specs, out_shape)(
        r, h, c, p['Wih'], p['bih'].reshape(1, -1), p['Whh'],
        p['bhh'].reshape(1, -1))


def _s2s_step(x, seg, h, bs, nrows):
    seg1 = seg[:nrows, 0]
    e, m = _s2s_pass1(x, seg, h, bs)
    w = _s2s_w(seg, e, m, bs)
    s = jax.ops.segment_sum(w[:nrows, 0], seg1, num_segments=B)
    ax = _s2s_ax(x, seg, w, s.reshape(1, B), bs)
    return jax.ops.segment_sum(ax[:nrows], seg1, num_segments=B)


def _set2set(x, seg, p, bs, nrows):
    z64 = jnp.zeros((B, 64), jnp.float32)
    h, c, _ = _s2s_lstm(z64, z64, z64, p, True)
    r = _s2s_step(x, seg, h, bs, nrows)
    for _ in range(2):
        h, c, _ = _s2s_lstm(r, h, c, p, False)
        r = _s2s_step(x, seg, h, bs, nrows)
    return h, r


def _head_body(hx_ref, rx_ref, he_ref, re_ref, gmx_ref, gapx_ref, gme_ref,
               gape_ref, aw1, ab1, aw2, ab2, l1w, l1b, l2w, l2b, o_ref):
    cat = jnp.concatenate(
        [hx_ref[...], rx_ref[...], he_ref[...], re_ref[...],
         gmx_ref[...], gapx_ref[...], gme_ref[...], gape_ref[...]], axis=1)
    a = jax.nn.relu(cat @ aw1[...] + ab1[...]) @ aw2[...] + ab2[...]
    out = jax.nn.softmax(a, axis=1) * cat * 512.0
    out = jax.nn.relu(out @ l1w[...] + l1b[...])
    o_ref[...] = out @ l2w[...] + l2b[...]


def _head(hx, rx, he, re, gmx, gapx, gme, gape, params):
    att, l1, l2 = params['att'], params['lin1'], params['lin2']
    in_specs = [_full((B, 64)), _full((B, 64)), _full((B, 64)), _full((B, 64)),
                _full((B, 64)), _full((B, 64)), _full((B, 64)), _full((B, 64)),
                _full((512, 512)), _full((1, 512)), _full((512, 512)),
                _full((1, 512)), _full((512, 128)), _full((1, 128)),
                _full((128, 2)), _full((1, 2))]
    return _pcall(_head_body, (1,), in_specs, _full((B, 2)),
                  jax.ShapeDtypeStruct((B, 2), jnp.float32))(
        hx, rx, he, re, gmx, gapx, gme, gape,
        att['W1'], att['b1'].reshape(1, -1), att['W2'], att['b2'].reshape(1, -1),
        l1['W'], l1['b'].reshape(1, -1), l2['W'], l2['b'].reshape(1, -1))


_SC_NW = 32
_SC_PER_W = EP // _SC_NW
_SC_CH = 128
_SC_NCH = _SC_PER_W // _SC_CH


def _sc_gather_body(F, table_hbm, idx_hbm, out_hbm, idx_v, rows_v, sem):
    wid = lax.axis_index("s") * 2 + lax.axis_index("c")
    base = wid * _SC_PER_W
    pltpu.sync_copy(idx_hbm.at[pl.ds(base, _SC_PER_W)], idx_v)

    def body(j, carry):
        pltpu.async_copy(
            table_hbm.at[idx_v.at[pl.ds(j * _SC_CH, _SC_CH)]], rows_v,
            sem).wait()
        pltpu.sync_copy(rows_v, out_hbm.at[pl.ds(base + j * _SC_CH, _SC_CH)])
        return carry

    lax.fori_loop(0, _SC_NCH, body, 0)


def _sc_gather(table, idx):
    F = table.shape[1]
    k = functools.partial(
        pl.kernel,
        mesh=plsc.VectorSubcoreMesh(core_axis_name="c", subcore_axis_name="s"),
        out_type=jax.ShapeDtypeStruct((EP, F), jnp.float32),
        scratch_types=[
            pltpu.VMEM((_SC_PER_W,), jnp.int32),
            pltpu.VMEM((_SC_CH, F), jnp.float32),
            pltpu.SemaphoreType.DMA,
        ],
    )(functools.partial(_sc_gather_body, F))
    return k(table, idx)


def _gather(table, idx):
    return _sc_gather(table, idx)


def _scatter_add(msg, dst, nrows):
    agg = jax.ops.segment_sum(msg, dst, num_segments=nrows)
    return agg[None]


def kernel(x, edge_attr, params, edge_index, x_batch, edge_attr_batch):
    f32 = jnp.float32
    src = jnp.pad(edge_index[0], (0, EP - E))
    dst = jnp.pad(edge_index[1], (0, EP - E))
    x0w = jnp.zeros((NP, 128), f32).at[:N, :9].set(x)
    x0 = x0w[:, 0:16]
    ea0 = jnp.zeros((EP, 8), f32).at[:E, :3].set(edge_attr)
    xseg = jnp.full((NP, 1), B, jnp.int32).at[:N, 0].set(x_batch)
    eseg = jnp.full((EP, 1), B, jnp.int32).at[:E, 0].set(edge_attr_batch)

    p = params
    l1 = p['layer1']
    w1p = (jnp.zeros((24, 128), f32).at[0:9].set(l1['W1'][0:9])
           .at[16:19].set(l1['W1'][9:12]))
    xs0 = _gather(x0w, src)
    msg = _msg1(xs0, ea0, w1p, l1['b1'].reshape(1, -1), l1['W2'],
                l1['b2'].reshape(1, -1))
    aggs = _scatter_add(msg, dst, NP)
    wr1 = jnp.zeros((16, 64), f32).at[:9].set(l1['Wr'])
    xk, xg = _node_update(x0, aggs, wr1, l1['br'].reshape(1, -1),
                          p['att_node1'])

    ea_g = ea0
    for k in (1, 2, 3):
        conv = p[f'layer{k + 1}']
        cat = jnp.concatenate([xk, xg], axis=1)
        gs = _gather(cat, src)
        gd = _gather(cat, dst)
        ea_g, msg = _edge_round(
            gs, gd, ea_g, p[f'eu{k}'], p[f'att_edge{k}'],
            conv['W1'], conv['b1'].reshape(1, -1),
            conv['W2'], conv['b2'].reshape(1, -1))
        aggs = _scatter_add(msg, dst, NP)
        if k < 3:
            xk, xg = _node_update(xg, aggs, conv['Wr'],
                                  conv['br'].reshape(1, -1),
                                  p[f'att_node{k + 1}'])
        else:
            xk = _node_update(xg, aggs, conv['Wr'], conv['br'].reshape(1, -1))
    cat4 = jnp.concatenate([xk, xk], axis=1)
    gs4 = _gather(cat4, src)
    gd4 = _gather(cat4, dst)
    ea4 = _edge_final(gs4, gd4, ea_g, p['eu4'])

    gmp_x = _pool_stats(xk, xseg, RN)
    gmp_e = _pool_stats(ea4, eseg, RE)
    gap_x = _gap(xk, xseg[:, 0], N)
    gap_e = _gap(ea4, eseg[:, 0], E)
    hx, rx = _set2set(xk, xseg, p['set2set'], RN, N)
    he, re = _set2set(ea4, eseg, p['set2set'], RE, E)
    return _head(hx, rx, he, re, gmp_x, gap_x, gmp_e, gap_e, p)
